# baseline (speedup 1.0000x reference)
import functools

import numpy as np

import jax
import jax.numpy as jnp
from jax import lax
from jax.experimental import pallas as pl
from jax.experimental.pallas import tpu as pltpu
from jax.experimental.pallas import tpu_sc as plsc

P = 14
S = 2
R = 1024
C = 256
NW = 32
BINS = R * P * P
BPW = BINS // NW
CHUNK = 8
NCH = BPW // CHUNK
NPAIR = NCH // 2
TAPS = 16
LVL_HW = (128, 64, 32, 16)
LVL_BASE = (0, 2 * 128 * 128, 2 * (128 * 128 + 64 * 64),
            2 * (128 * 128 + 64 * 64 + 32 * 32))
TABLE_ROWS = 2 * sum(h * h for h in LVL_HW)

_q = np.arange(P * P * 2 * 2 * 2 * 2)
_b = _q % 2
_a = (_q // 2) % 2
_sj = (_q // 4) % 2
_si = (_q // 8) % 2
_px = (_q // 16) % P
_py = _q // (16 * P)
_ky = (_py * 2 + _si) * 2 + _a
_kx = (_px * 2 + _sj) * 2 + _b
_SEL_Y = (_ky[None, :] == np.arange(4 * P)[:, None]).astype(np.float32)
_SEL_X = (_kx[None, :] == np.arange(4 * P)[:, None]).astype(np.float32)


def _build_table(x0, x1, x2, x3):
    rows = []
    for x in (x0, x1, x2, x3):
        bits = jax.lax.bitcast_convert_type(
            x.astype(jnp.bfloat16), jnp.uint16).astype(jnp.uint32)
        word = (bits[:, C // 2:] << 16) | bits[:, : C // 2]
        word = jnp.transpose(word.astype(jnp.int32), (0, 2, 3, 1))
        rows.append(word.reshape(-1, C // 2))
    return jnp.concatenate(rows, axis=0)


def _build_taps(boxes, box_batch_idx):
    f32 = jnp.float32
    i32 = jnp.int32
    bw = boxes[:, 2] - boxes[:, 0]
    bh = boxes[:, 3] - boxes[:, 1]
    s = jnp.sqrt(bw * bh)
    lvl = jnp.clip(jnp.floor(4.0 + jnp.log2(s / 224.0 + 1e-6)), 2, 5).astype(i32)
    li = lvl - 2
    scale = jnp.take(jnp.array([0.25, 0.125, 0.0625, 0.03125], f32), li)
    hw = jnp.take(jnp.array(LVL_HW, i32), li)
    base = jnp.take(jnp.array(LVL_BASE, i32), li)
    hwf = hw.astype(f32)

    sx1 = boxes[:, 0] * scale
    sy1 = boxes[:, 1] * scale
    sx2 = boxes[:, 2] * scale
    sy2 = boxes[:, 3] * scale
    roi_w = jnp.maximum(sx2 - sx1, 1.0)
    roi_h = jnp.maximum(sy2 - sy1, 1.0)
    bin_w = roi_w / P
    bin_h = roi_h / P

    gv = (jnp.arange(P, dtype=f32)[:, None]
          + (jnp.arange(S, dtype=f32)[None, :] + 0.5) / S)
    ys = sy1[:, None, None] + bin_h[:, None, None] * gv[None]
    xs = sx1[:, None, None] + bin_w[:, None, None] * gv[None]
    vy = ((ys >= -1.0) & (ys <= hwf[:, None, None])).astype(f32)
    vx = ((xs >= -1.0) & (xs <= hwf[:, None, None])).astype(f32)
    yc = jnp.clip(ys, 0.0, hwf[:, None, None] - 1.0)
    xc = jnp.clip(xs, 0.0, hwf[:, None, None] - 1.0)
    y_lo = jnp.clip(jnp.floor(yc).astype(i32), 0, hw[:, None, None] - 1)
    y_hi = jnp.minimum(y_lo + 1, hw[:, None, None] - 1)
    x_lo = jnp.clip(jnp.floor(xc).astype(i32), 0, hw[:, None, None] - 1)
    x_hi = jnp.minimum(x_lo + 1, hw[:, None, None] - 1)
    ly = yc - y_lo.astype(f32)
    lx = xc - x_lo.astype(f32)

    rowbase = base + box_batch_idx.astype(i32) * hw * hw
    yy = (jnp.stack([y_lo, y_hi], axis=-1) * hw[:, None, None, None]
          + rowbase[:, None, None, None])
    wy = jnp.stack([1.0 - ly, ly], axis=-1) * vy[..., None]
    xx = jnp.stack([x_lo, x_hi], axis=-1)
    wx = jnp.stack([1.0 - lx, lx], axis=-1) * (vx[..., None] * (1.0 / (S * S)))

    hp = jax.lax.Precision.HIGHEST
    sel_y = jnp.asarray(_SEL_Y)
    sel_x = jnp.asarray(_SEL_X)
    tap = (jnp.matmul(yy.reshape(R, 4 * P).astype(f32), sel_y, precision=hp)
           + jnp.matmul(xx.reshape(R, 4 * P).astype(f32), sel_x, precision=hp))
    tw = (jnp.matmul(wy.reshape(R, 4 * P), sel_y, precision=hp)
          * jnp.matmul(wx.reshape(R, 4 * P), sel_x, precision=hp))
    idx_arr = tap.astype(i32).reshape(NW, NCH, CHUNK * TAPS)
    w_arr = tw.reshape(NW, NCH, CHUNK * TAPS)
    return idx_arr, w_arr


def _reduce_chunk(rows, w_v, half, outb):
    def one_bin(b):
        acc_e = [jnp.zeros((16,), jnp.float32) for _ in range(C // 32)]
        acc_o = [jnp.zeros((16,), jnp.float32) for _ in range(C // 32)]
        w_vec = w_v[half, pl.ds(b * TAPS, TAPS)]
        for t in range(TAPS):
            w_s = w_vec[t]
            for g in range(C // 32):
                v = rows[b * TAPS + t, pl.ds(16 * g, 16)]
                ve = lax.bitcast_convert_type(jnp.left_shift(v, 16), jnp.float32)
                vo = lax.bitcast_convert_type(v, jnp.float32)
                acc_e[g] = acc_e[g] + ve * w_s
                acc_o[g] = acc_o[g] + vo * w_s
        for g in range(C // 32):
            outb[pl.ds(b * C + 16 * g, 16)] = acc_e[g]
            outb[pl.ds(b * C + C // 2 + 16 * g, 16)] = acc_o[g]

    def bin_body(m, carry):
        one_bin(2 * m)
        one_bin(2 * m + 1)
        return carry
    lax.fori_loop(0, CHUNK // 2, bin_body, 0)


@functools.lru_cache(maxsize=1)
def _make_sc_pool():
    @functools.partial(
        pl.kernel,
        out_type=jax.ShapeDtypeStruct((BINS * C,), jnp.float32),
        mesh=plsc.VectorSubcoreMesh(core_axis_name="c", subcore_axis_name="s"),
        scratch_types=[
            pltpu.VMEM((2, CHUNK * TAPS), jnp.int32),
            pltpu.VMEM((2, CHUNK * TAPS), jnp.float32),
            pltpu.VMEM((CHUNK * TAPS, C // 2), jnp.int32),
            pltpu.VMEM((CHUNK * TAPS, C // 2), jnp.int32),
            pltpu.VMEM((CHUNK * C,), jnp.float32),
            pltpu.VMEM((CHUNK * C,), jnp.float32),
            pltpu.SemaphoreType.DMA,
            pltpu.SemaphoreType.DMA,
            pltpu.SemaphoreType.DMA,
            pltpu.SemaphoreType.DMA,
            pltpu.SemaphoreType.DMA,
            pltpu.SemaphoreType.DMA,
            pltpu.SemaphoreType.DMA,
            pltpu.SemaphoreType.DMA,
        ],
    )
    def _sc_pool(table, idxs, ws, out, idx_v, w_v, rows0, rows1,
                 outb0, outb1, sem0, sem1, ssem0, ssem1,
                 isem0, isem1, wsem0, wsem1):
        cid = lax.axis_index("c")
        sid = lax.axis_index("s")
        wid = sid * 2 + cid
        obase = wid * BPW * C

        def ostore(outb, chunk, ssem):
            return pltpu.async_copy(
                outb, out.at[pl.ds(obase + chunk * CHUNK * C, CHUNK * C)], ssem)

        def owait(outb, ssem):
            pltpu.make_async_copy(
                outb, out.at[pl.ds(obase, CHUNK * C)], ssem).wait()

        pltpu.sync_copy(idxs.at[wid, pl.ds(0, 2)], idx_v)
        pltpu.sync_copy(ws.at[wid, pl.ds(0, 2)], w_v)
        pltpu.async_copy(table.at[idx_v.at[0]], rows0, sem0)
        pltpu.async_copy(table.at[idx_v.at[1]], rows1, sem1)

        def half_body(k, c0, half, rows, sem, outb, ssem, isem, wsem):
            pltpu.make_async_copy(table.at[idx_v.at[half]], rows, sem).wait()

            @pl.when(k < NPAIR - 1)
            def _():
                pltpu.async_copy(idxs.at[wid, c0 + 2], idx_v.at[half], isem)

            @pl.when(k > 0)
            def _():
                owait(outb, ssem)
                pltpu.make_async_copy(
                    ws.at[wid, c0], w_v.at[half], wsem).wait()
            _reduce_chunk(rows, w_v, half, outb)
            ostore(outb, c0, ssem)

            @pl.when(k < NPAIR - 1)
            def _():
                pltpu.make_async_copy(
                    idxs.at[wid, c0 + 2], idx_v.at[half], isem).wait()
                pltpu.async_copy(table.at[idx_v.at[half]], rows, sem)
                pltpu.async_copy(ws.at[wid, c0 + 2], w_v.at[half], wsem)

        def pair_body(k, carry):
            c0 = 2 * k
            half_body(k, c0, 0, rows0, sem0, outb0, ssem0, isem0, wsem0)
            half_body(k, c0 + 1, 1, rows1, sem1, outb1, ssem1, isem1, wsem1)
            return carry

        lax.fori_loop(0, NPAIR, pair_body, 0)
        owait(outb0, ssem0)
        owait(outb1, ssem1)

    return _sc_pool


def kernel(x0, x1, x2, x3, boxes, box_batch_idx):
    table = _build_table(x0, x1, x2, x3)
    idx_arr, w_arr = _build_taps(boxes, box_batch_idx)
    out = _make_sc_pool()(table, idx_arr, w_arr)
    eye = jnp.eye(P * P, dtype=jnp.float32)
    out_t = jnp.einsum("rkc,kb->rcb", out.reshape(R, P * P, C), eye)
    return out_t.reshape(R, C, P, P)


# speedup vs baseline: 1.1219x; 1.1219x over previous
import functools

import numpy as np

import jax
import jax.numpy as jnp
from jax import lax
from jax.experimental import pallas as pl
from jax.experimental.pallas import tpu as pltpu
from jax.experimental.pallas import tpu_sc as plsc

P = 14
S = 2
R = 1024
C = 256
NW = 32
BINS = R * P * P
BPW = BINS // NW
CHUNK = 8
NCH = BPW // CHUNK
NBUF = 4
NQUAD = NCH // NBUF
TAPS = 16
LVL_HW = (128, 64, 32, 16)
LVL_BASE = (0, 2 * 128 * 128, 2 * (128 * 128 + 64 * 64),
            2 * (128 * 128 + 64 * 64 + 32 * 32))
TABLE_ROWS = 2 * sum(h * h for h in LVL_HW)

_q = np.arange(P * P * 2 * 2 * 2 * 2)
_b = _q % 2
_a = (_q // 2) % 2
_sj = (_q // 4) % 2
_si = (_q // 8) % 2
_px = (_q // 16) % P
_py = _q // (16 * P)
_ky = (_py * 2 + _si) * 2 + _a
_kx = (_px * 2 + _sj) * 2 + _b
_SEL_Y = (_ky[None, :] == np.arange(4 * P)[:, None]).astype(np.float32)
_SEL_X = (_kx[None, :] == np.arange(4 * P)[:, None]).astype(np.float32)


def _build_table(x0, x1, x2, x3):
    rows = []
    for x in (x0, x1, x2, x3):
        bits = jax.lax.bitcast_convert_type(
            x.astype(jnp.bfloat16), jnp.uint16).astype(jnp.uint32)
        word = (bits[:, C // 2:] << 16) | bits[:, : C // 2]
        word = jnp.transpose(word.astype(jnp.int32), (0, 2, 3, 1))
        rows.append(word.reshape(-1, C // 2))
    return jnp.concatenate(rows, axis=0)


def _build_taps(boxes, box_batch_idx):
    f32 = jnp.float32
    i32 = jnp.int32
    bw = boxes[:, 2] - boxes[:, 0]
    bh = boxes[:, 3] - boxes[:, 1]
    s = jnp.sqrt(bw * bh)
    lvl = jnp.clip(jnp.floor(4.0 + jnp.log2(s / 224.0 + 1e-6)), 2, 5).astype(i32)
    li = lvl - 2
    scale = jnp.take(jnp.array([0.25, 0.125, 0.0625, 0.03125], f32), li)
    hw = jnp.take(jnp.array(LVL_HW, i32), li)
    base = jnp.take(jnp.array(LVL_BASE, i32), li)
    hwf = hw.astype(f32)

    sx1 = boxes[:, 0] * scale
    sy1 = boxes[:, 1] * scale
    sx2 = boxes[:, 2] * scale
    sy2 = boxes[:, 3] * scale
    roi_w = jnp.maximum(sx2 - sx1, 1.0)
    roi_h = jnp.maximum(sy2 - sy1, 1.0)
    bin_w = roi_w / P
    bin_h = roi_h / P

    gv = (jnp.arange(P, dtype=f32)[:, None]
          + (jnp.arange(S, dtype=f32)[None, :] + 0.5) / S)
    ys = sy1[:, None, None] + bin_h[:, None, None] * gv[None]
    xs = sx1[:, None, None] + bin_w[:, None, None] * gv[None]
    vy = ((ys >= -1.0) & (ys <= hwf[:, None, None])).astype(f32)
    vx = ((xs >= -1.0) & (xs <= hwf[:, None, None])).astype(f32)
    yc = jnp.clip(ys, 0.0, hwf[:, None, None] - 1.0)
    xc = jnp.clip(xs, 0.0, hwf[:, None, None] - 1.0)
    y_lo = jnp.clip(jnp.floor(yc).astype(i32), 0, hw[:, None, None] - 1)
    y_hi = jnp.minimum(y_lo + 1, hw[:, None, None] - 1)
    x_lo = jnp.clip(jnp.floor(xc).astype(i32), 0, hw[:, None, None] - 1)
    x_hi = jnp.minimum(x_lo + 1, hw[:, None, None] - 1)
    ly = yc - y_lo.astype(f32)
    lx = xc - x_lo.astype(f32)

    rowbase = base + box_batch_idx.astype(i32) * hw * hw
    yy = (jnp.stack([y_lo, y_hi], axis=-1) * hw[:, None, None, None]
          + rowbase[:, None, None, None])
    wy = jnp.stack([1.0 - ly, ly], axis=-1) * vy[..., None]
    xx = jnp.stack([x_lo, x_hi], axis=-1)
    wx = jnp.stack([1.0 - lx, lx], axis=-1) * (vx[..., None] * (1.0 / (S * S)))

    hp = jax.lax.Precision.HIGHEST
    sel_y = jnp.asarray(_SEL_Y)
    sel_x = jnp.asarray(_SEL_X)
    tap = (jnp.matmul(yy.reshape(R, 4 * P).astype(f32), sel_y, precision=hp)
           + jnp.matmul(xx.reshape(R, 4 * P).astype(f32), sel_x, precision=hp))
    tw = (jnp.matmul(wy.reshape(R, 4 * P), sel_y, precision=hp)
          * jnp.matmul(wx.reshape(R, 4 * P), sel_x, precision=hp))
    idx_arr = tap.astype(i32).reshape(NW, NCH, CHUNK * TAPS)
    w_arr = tw.reshape(NW, NCH, CHUNK * TAPS)
    return idx_arr, w_arr


def _reduce_chunk(rows, w_v, half, outb):
    def one_bin(b):
        acc_e = [jnp.zeros((16,), jnp.float32) for _ in range(C // 32)]
        acc_o = [jnp.zeros((16,), jnp.float32) for _ in range(C // 32)]
        w_vec = w_v[half, pl.ds(b * TAPS, TAPS)]
        for t in range(TAPS):
            w_s = w_vec[t]
            for g in range(C // 32):
                v = rows[b * TAPS + t, pl.ds(16 * g, 16)]
                ve = lax.bitcast_convert_type(jnp.left_shift(v, 16), jnp.float32)
                vo = lax.bitcast_convert_type(v, jnp.float32)
                acc_e[g] = acc_e[g] + ve * w_s
                acc_o[g] = acc_o[g] + vo * w_s
        for g in range(C // 32):
            outb[pl.ds(b * C + 16 * g, 16)] = acc_e[g]
            outb[pl.ds(b * C + C // 2 + 16 * g, 16)] = acc_o[g]

    def bin_body(m, carry):
        one_bin(2 * m)
        one_bin(2 * m + 1)
        return carry
    lax.fori_loop(0, CHUNK // 2, bin_body, 0)


@functools.lru_cache(maxsize=1)
def _make_sc_pool():
    @functools.partial(
        pl.kernel,
        out_type=jax.ShapeDtypeStruct((BINS * C,), jnp.float32),
        mesh=plsc.VectorSubcoreMesh(core_axis_name="c", subcore_axis_name="s"),
        scratch_types=(
            [pltpu.VMEM((NBUF, CHUNK * TAPS), jnp.int32),
             pltpu.VMEM((NBUF, CHUNK * TAPS), jnp.float32)]
            + [pltpu.VMEM((CHUNK * TAPS, C // 2), jnp.int32)] * NBUF
            + [pltpu.VMEM((CHUNK * C,), jnp.float32)] * 2
            + [pltpu.SemaphoreType.DMA] * (2 + 3 * NBUF)
        ),
    )
    def _sc_pool(table, idxs, ws, out, idx_v, w_v, *rest):
        rows = rest[:NBUF]
        outb = rest[NBUF:NBUF + 2]
        ssem = rest[NBUF + 2:NBUF + 4]
        sem = rest[NBUF + 4:NBUF + 4 + NBUF]
        isem = rest[NBUF + 4 + NBUF:NBUF + 4 + 2 * NBUF]
        wsem = rest[NBUF + 4 + 2 * NBUF:]
        cid = lax.axis_index("c")
        sid = lax.axis_index("s")
        wid = sid * 2 + cid
        obase = wid * BPW * C

        def ostore(ob, chunk, ss):
            return pltpu.async_copy(
                ob, out.at[pl.ds(obase + chunk * CHUNK * C, CHUNK * C)], ss)

        def owait(ob, ss):
            pltpu.make_async_copy(
                ob, out.at[pl.ds(obase, CHUNK * C)], ss).wait()

        pltpu.sync_copy(idxs.at[wid, pl.ds(0, NBUF)], idx_v)
        pltpu.sync_copy(ws.at[wid, pl.ds(0, NBUF)], w_v)
        for j in range(NBUF):
            pltpu.async_copy(table.at[idx_v.at[j]], rows[j], sem[j])

        def quad_body(k, carry):
            c_base = NBUF * k
            for j in range(NBUF):
                c0 = c_base + j
                pltpu.make_async_copy(
                    table.at[idx_v.at[j]], rows[j], sem[j]).wait()

                @pl.when(k < NQUAD - 1)
                def _():
                    pltpu.async_copy(
                        idxs.at[wid, c0 + NBUF], idx_v.at[j], isem[j])

                if j >= 2:
                    owait(outb[j % 2], ssem[j % 2])
                else:
                    @pl.when(k > 0)
                    def _():
                        owait(outb[j % 2], ssem[j % 2])

                @pl.when(k > 0)
                def _():
                    pltpu.make_async_copy(
                        ws.at[wid, c0], w_v.at[j], wsem[j]).wait()
                _reduce_chunk(rows[j], w_v, j, outb[j % 2])
                ostore(outb[j % 2], c0, ssem[j % 2])

                @pl.when(k < NQUAD - 1)
                def _():
                    pltpu.make_async_copy(
                        idxs.at[wid, c0 + NBUF], idx_v.at[j], isem[j]).wait()
                    pltpu.async_copy(table.at[idx_v.at[j]], rows[j], sem[j])
                    pltpu.async_copy(ws.at[wid, c0 + NBUF], w_v.at[j], wsem[j])

            return carry

        lax.fori_loop(0, NQUAD, quad_body, 0)
        owait(outb[0], ssem[0])
        owait(outb[1], ssem[1])

    return _sc_pool


def kernel(x0, x1, x2, x3, boxes, box_batch_idx):
    table = _build_table(x0, x1, x2, x3)
    idx_arr, w_arr = _build_taps(boxes, box_batch_idx)
    out = _make_sc_pool()(table, idx_arr, w_arr)
    return out.reshape(R, P, P, C).transpose(0, 3, 1, 2)
